# baseline (speedup 1.0000x reference)
import functools

import jax
import jax.numpy as jnp
from jax.experimental import pallas as pl
from jax.experimental.pallas import tpu as pltpu
from jax.sharding import PartitionSpec as P
from jax.experimental.shard_map import shard_map


def _gather_ce_kernel(idx_ref, tgt_ref, table_ref, logits_ref, aux_ref,
                      rows_ref, packed_ref, *, tile_n, v_loc):
    w = v_loc // 2
    for mi in range(tile_n):
        rows_ref[mi, 0] = table_ref[idx_ref[0, 0, mi], 0]

    packed_ref[...] = rows_ref[...].reshape(tile_n, w)

    x = packed_ref[...]
    left = jax.lax.bitcast_convert_type(x & jnp.int32(-65536), jnp.float32)
    right = jax.lax.bitcast_convert_type(x << 16, jnp.float32)
    logits_ref[:, :w] = left
    logits_ref[:, w:] = right

    vals = logits_ref[...]
    col = jax.lax.broadcasted_iota(jnp.int32, (tile_n, v_loc), 1)
    m = jnp.max(vals, axis=-1, keepdims=True)
    s = jnp.sum(jnp.exp(vals - m), axis=-1, keepdims=True)
    t = jnp.sum(jnp.where(col == tgt_ref[...], vals, 0.0),
                axis=-1, keepdims=True)
    aux_ref[...] = jnp.concatenate([m, s, t], axis=-1)


def _forward_local(idx_rows, tgt_loc, table_packed, *, tile_n, v_loc):
    num_tiles = idx_rows.shape[0]
    n_loc = num_tiles * tile_n
    v = table_packed.shape[0]
    w = v_loc // 2

    body = functools.partial(_gather_ce_kernel, tile_n=tile_n, v_loc=v_loc)
    return pl.pallas_call(
        body,
        grid=(num_tiles,),
        out_shape=(
            jax.ShapeDtypeStruct((n_loc, v_loc), jnp.float32),
            jax.ShapeDtypeStruct((n_loc, 3), jnp.float32),
        ),
        in_specs=[
            pl.BlockSpec((1, 1, tile_n), lambda i: (i, 0, 0),
                         memory_space=pltpu.SMEM),
            pl.BlockSpec((tile_n, 1), lambda i: (i, 0)),
            pl.BlockSpec((v, 1, w), lambda i: (0, 0, 0)),
        ],
        out_specs=(
            pl.BlockSpec((tile_n, v_loc), lambda i: (i, 0)),
            pl.BlockSpec((tile_n, 3), lambda i: (i, 0)),
        ),
        scratch_shapes=[pltpu.VMEM((tile_n, 1, w), jnp.int32),
                        pltpu.VMEM((tile_n, w), jnp.int32)],
        compiler_params=pltpu.CompilerParams(
            dimension_semantics=("parallel",)),
    )(idx_rows, tgt_loc, table_packed)


def kernel(idx, embedding_table, targets):
    B, T = idx.shape
    V = embedding_table.shape[0]
    N = B * T

    tile_n = 256
    assert N % tile_n == 0

    ndev = 2 if jax.local_device_count() >= 2 else 1
    C = V // ndev
    W = C // 2
    assert C % 256 == 0

    tb = embedding_table.astype(jnp.bfloat16)
    hi_cols = jnp.concatenate([tb[:, C * d: C * d + W] for d in range(ndev)],
                              axis=1)
    lo_cols = jnp.concatenate([tb[:, C * d + W: C * (d + 1)]
                               for d in range(ndev)], axis=1)
    hi = jax.lax.bitcast_convert_type(hi_cols, jnp.uint16).astype(jnp.uint32)
    lo = jax.lax.bitcast_convert_type(lo_cols, jnp.uint16).astype(jnp.uint32)
    packed = jax.lax.bitcast_convert_type(
        (hi << 16) | lo, jnp.int32).reshape(V, 1, V // 2)

    mesh = jax.make_mesh((ndev,), ("x",))

    def shard_fn(idx_s, tgt_s, packed_s):
        d = jax.lax.axis_index("x")
        num_tiles = N // tile_n
        idx_rows = idx_s.reshape(num_tiles, 1, tile_n).astype(jnp.int32)
        tgt_loc = (tgt_s.reshape(N, 1).astype(jnp.int32) - d * C)
        logits_l, aux_l = _forward_local(
            idx_rows, tgt_loc, packed_s, tile_n=tile_n, v_loc=C)

        aux_all = jax.lax.all_gather(aux_l, "x")
        m_all = aux_all[:, :, 0]
        s_all = aux_all[:, :, 1]
        t_all = aux_all[:, :, 2]
        m_glob = jnp.max(m_all, axis=0)
        s_glob = jnp.sum(s_all * jnp.exp(m_all - m_glob[None, :]), axis=0)
        lse = m_glob + jnp.log(s_glob)
        tok_loss = lse - jnp.sum(t_all, axis=0)
        return logits_l, jnp.sum(tok_loss)

    sharded = shard_map(
        shard_fn, mesh=mesh,
        in_specs=(P(None, None), P(None, None), P(None, None, "x")),
        out_specs=(P(None, "x"), P()),
        check_rep=False,
    )
    repl = jax.sharding.NamedSharding(mesh, P(None, None))
    idx = jax.reshard(idx, repl)
    targets = jax.reshard(targets, repl)
    packed = jax.reshard(
        packed, jax.sharding.NamedSharding(mesh, P(None, None, "x")))
    logits, loss_sum = sharded(idx, targets, packed)
    return logits, loss_sum / N


# speedup vs baseline: 2.0514x; 2.0514x over previous
import functools

import jax
import jax.numpy as jnp
from jax.experimental import pallas as pl
from jax.experimental.pallas import tpu as pltpu
from jax.sharding import PartitionSpec as P
from jax.experimental.shard_map import shard_map


def _gather_ce_kernel(idx_ref, tgt_ref, table_ref, logits_ref, tokloss_ref,
                      rows_ref, packed_ref, *, tile_n, v):
    w = v // 2
    for mi in range(tile_n):
        rows_ref[mi, 0] = table_ref[idx_ref[0, 0, mi], 0]

    packed_ref[...] = rows_ref[...].reshape(tile_n, w)

    x = packed_ref[...]
    left = jax.lax.bitcast_convert_type(x & jnp.int32(-65536), jnp.float32)
    right = jax.lax.bitcast_convert_type(x << 16, jnp.float32)
    logits_ref[:, :w] = left
    logits_ref[:, w:] = right

    vals = logits_ref[...]
    col = jax.lax.broadcasted_iota(jnp.int32, (tile_n, v), 1)
    m = jnp.max(vals, axis=-1, keepdims=True)
    lse = m + jnp.log(jnp.sum(jnp.exp(vals - m), axis=-1, keepdims=True))
    tgt_logit = jnp.sum(jnp.where(col == tgt_ref[...], vals, 0.0),
                        axis=-1, keepdims=True)
    tokloss_ref[...] = lse - tgt_logit


def _forward_local(idx_part, tgt_part, table_packed, *, tile_n, v):
    n_loc = idx_part.size
    num_tiles = n_loc // tile_n
    w = v // 2

    idx_rows = idx_part.reshape(num_tiles, 1, tile_n).astype(jnp.int32)
    tgt_col = tgt_part.reshape(n_loc, 1).astype(jnp.int32)

    body = functools.partial(_gather_ce_kernel, tile_n=tile_n, v=v)
    return pl.pallas_call(
        body,
        grid=(num_tiles,),
        out_shape=(
            jax.ShapeDtypeStruct((n_loc, v), jnp.float32),
            jax.ShapeDtypeStruct((n_loc, 1), jnp.float32),
        ),
        in_specs=[
            pl.BlockSpec((1, 1, tile_n), lambda i: (i, 0, 0),
                         memory_space=pltpu.SMEM),
            pl.BlockSpec((tile_n, 1), lambda i: (i, 0)),
            pl.BlockSpec((v, 1, w), lambda i: (0, 0, 0)),
        ],
        out_specs=(
            pl.BlockSpec((tile_n, v), lambda i: (i, 0)),
            pl.BlockSpec((tile_n, 1), lambda i: (i, 0)),
        ),
        scratch_shapes=[pltpu.VMEM((tile_n, 1, w), jnp.int32),
                        pltpu.VMEM((tile_n, w), jnp.int32)],
        compiler_params=pltpu.CompilerParams(
            dimension_semantics=("parallel",)),
    )(idx_rows, tgt_col, table_packed)


def kernel(idx, embedding_table, targets):
    B, T = idx.shape
    V = embedding_table.shape[0]
    N = B * T
    W = V // 2

    tile_n = 256
    assert V % 256 == 0

    ndev = 2 if jax.local_device_count() >= 2 else 1
    assert (N // ndev) % tile_n == 0 and B % ndev == 0

    tb = embedding_table.astype(jnp.bfloat16)
    hi = jax.lax.bitcast_convert_type(tb[:, :W], jnp.uint16).astype(jnp.uint32)
    lo = jax.lax.bitcast_convert_type(tb[:, W:], jnp.uint16).astype(jnp.uint32)
    packed = jax.lax.bitcast_convert_type(
        (hi << 16) | lo, jnp.int32).reshape(V, 1, W)
    packed = jax.lax.optimization_barrier(packed)

    mesh = jax.make_mesh((ndev,), ("x",))

    def shard_fn(idx_s, tgt_s, packed_s):
        logits_l, tokloss_l = _forward_local(
            idx_s, tgt_s, packed_s, tile_n=tile_n, v=V)
        loss_sum = jax.lax.psum(jnp.sum(tokloss_l), "x")
        return logits_l, loss_sum

    sharded = shard_map(
        shard_fn, mesh=mesh,
        in_specs=(P("x"), P("x"), P(None, None, None)),
        out_specs=(P("x"), P()),
        check_rep=False,
    )
    idx = jax.reshard(idx, jax.sharding.NamedSharding(mesh, P("x")))
    targets = jax.reshard(targets, jax.sharding.NamedSharding(mesh, P("x")))
    packed = jax.reshard(
        packed, jax.sharding.NamedSharding(mesh, P(None, None, None)))
    logits, loss_sum = sharded(idx, targets, packed)
    return logits, loss_sum / N


# speedup vs baseline: 2.4762x; 1.2071x over previous
import functools

import jax
import jax.numpy as jnp
from jax.experimental import pallas as pl
from jax.experimental.pallas import tpu as pltpu


def _gather_ce_kernel(idx_ref, tgt_ref, table_ref, logits_ref, tokloss_ref,
                      rows_ref, packed_ref, *, tile_n, v):
    w = v // 2
    for mi in range(tile_n):
        rows_ref[mi, 0] = table_ref[idx_ref[0, 0, mi], 0]

    packed_ref[...] = rows_ref[...].reshape(tile_n, w)

    x = packed_ref[...]
    left = jax.lax.bitcast_convert_type(x & jnp.int32(-65536), jnp.float32)
    right = jax.lax.bitcast_convert_type(x << 16, jnp.float32)
    logits_ref[:, :w] = left
    logits_ref[:, w:] = right

    vals = logits_ref[...]
    col = jax.lax.broadcasted_iota(jnp.int32, (tile_n, v), 1)
    m = jnp.max(vals, axis=-1, keepdims=True)
    lse = m + jnp.log(jnp.sum(jnp.exp(vals - m), axis=-1, keepdims=True))
    tgt_logit = jnp.sum(jnp.where(col == tgt_ref[...], vals, 0.0),
                        axis=-1, keepdims=True)
    tokloss_ref[...] = lse - tgt_logit


def kernel(idx, embedding_table, targets):
    B, T = idx.shape
    V = embedding_table.shape[0]
    N = B * T
    W = V // 2

    tile_n = 256
    assert N % tile_n == 0 and V % 256 == 0
    num_tiles = N // tile_n

    tb = embedding_table.astype(jnp.bfloat16)
    hi = jax.lax.bitcast_convert_type(tb[:, :W], jnp.uint16).astype(jnp.uint32)
    lo = jax.lax.bitcast_convert_type(tb[:, W:], jnp.uint16).astype(jnp.uint32)
    packed = jax.lax.bitcast_convert_type(
        (hi << 16) | lo, jnp.int32).reshape(V, 1, W)

    idx_rows = idx.reshape(num_tiles, 1, tile_n).astype(jnp.int32)
    tgt_col = targets.reshape(N, 1).astype(jnp.int32)

    body = functools.partial(_gather_ce_kernel, tile_n=tile_n, v=V)
    logits, tok_loss = pl.pallas_call(
        body,
        grid=(num_tiles,),
        out_shape=(
            jax.ShapeDtypeStruct((N, V), jnp.float32),
            jax.ShapeDtypeStruct((N, 1), jnp.float32),
        ),
        in_specs=[
            pl.BlockSpec((1, 1, tile_n), lambda i: (i, 0, 0),
                         memory_space=pltpu.SMEM),
            pl.BlockSpec((tile_n, 1), lambda i: (i, 0)),
            pl.BlockSpec((V, 1, W), lambda i: (0, 0, 0)),
        ],
        out_specs=(
            pl.BlockSpec((tile_n, V), lambda i: (i, 0)),
            pl.BlockSpec((tile_n, 1), lambda i: (i, 0)),
        ),
        scratch_shapes=[pltpu.VMEM((tile_n, 1, W), jnp.int32),
                        pltpu.VMEM((tile_n, W), jnp.int32)],
        compiler_params=pltpu.CompilerParams(
            dimension_semantics=("parallel",)),
    )(idx_rows, tgt_col, packed)

    loss = jnp.sum(tok_loss) / N
    return logits, loss


# speedup vs baseline: 2.9208x; 1.1795x over previous
import functools

import jax
import jax.numpy as jnp
from jax.experimental import pallas as pl
from jax.experimental.pallas import tpu as pltpu


def _gather_ce_kernel(idx_ref, tgt_ref, table_ref, logits_ref, tokloss_ref,
                      rows_ref, packed_ref, *, tile_n, v):
    w = v // 2
    for mi in range(tile_n):
        rows_ref[mi, 0] = table_ref[idx_ref[0, 0, mi], 0]

    packed_ref[...] = rows_ref[...].reshape(tile_n, w)

    x = packed_ref[...]
    left = jax.lax.bitcast_convert_type(x & jnp.int32(-65536), jnp.float32)
    right = jax.lax.bitcast_convert_type(x << 16, jnp.float32)
    logits_ref[:, :w] = left
    logits_ref[:, w:] = right

    vals = logits_ref[...]
    col = jax.lax.broadcasted_iota(jnp.int32, (tile_n, v), 1)
    m = jnp.max(vals, axis=-1, keepdims=True)
    lse = m + jnp.log(jnp.sum(jnp.exp(vals - m), axis=-1, keepdims=True))
    tgt_logit = jnp.sum(jnp.where(col == tgt_ref[...], vals, 0.0),
                        axis=-1, keepdims=True)
    tokloss_ref[...] = lse - tgt_logit


def kernel(idx, embedding_table, targets):
    B, T = idx.shape
    V = embedding_table.shape[0]
    N = B * T
    W = V // 2

    tile_n = 512
    assert N % tile_n == 0 and V % 256 == 0
    num_tiles = N // tile_n

    tb = embedding_table.astype(jnp.bfloat16)
    hi = jax.lax.bitcast_convert_type(tb[:, :W], jnp.uint16).astype(jnp.uint32)
    lo = jax.lax.bitcast_convert_type(tb[:, W:], jnp.uint16).astype(jnp.uint32)
    packed = jax.lax.bitcast_convert_type(
        (hi << 16) | lo, jnp.int32).reshape(V, 1, W)

    idx_rows = idx.reshape(num_tiles, 1, tile_n).astype(jnp.int32)
    tgt_col = targets.reshape(N, 1).astype(jnp.int32)

    body = functools.partial(_gather_ce_kernel, tile_n=tile_n, v=V)
    logits, tok_loss = pl.pallas_call(
        body,
        grid=(num_tiles,),
        out_shape=(
            jax.ShapeDtypeStruct((N, V), jnp.float32),
            jax.ShapeDtypeStruct((N, 1), jnp.float32),
        ),
        in_specs=[
            pl.BlockSpec((1, 1, tile_n), lambda i: (i, 0, 0),
                         memory_space=pltpu.SMEM),
            pl.BlockSpec((tile_n, 1), lambda i: (i, 0)),
            pl.BlockSpec((V, 1, W), lambda i: (0, 0, 0)),
        ],
        out_specs=(
            pl.BlockSpec((tile_n, V), lambda i: (i, 0)),
            pl.BlockSpec((tile_n, 1), lambda i: (i, 0)),
        ),
        scratch_shapes=[pltpu.VMEM((tile_n, 1, W), jnp.int32),
                        pltpu.VMEM((tile_n, W), jnp.int32)],
        compiler_params=pltpu.CompilerParams(
            dimension_semantics=("parallel",)),
    )(idx_rows, tgt_col, packed)

    loss = jnp.sum(tok_loss) / N
    return logits, loss


# speedup vs baseline: 3.1547x; 1.0801x over previous
import functools

import jax
import jax.numpy as jnp
from jax.experimental import pallas as pl
from jax.experimental.pallas import tpu as pltpu


def _gather_ce_kernel(idx_ref, tgt_ref, table_ref, logits_ref, tokloss_ref,
                      rows_ref, packed_ref, *, tile_n, v):
    w = v // 2
    for mi in range(tile_n):
        rows_ref[mi, 0] = table_ref[idx_ref[0, 0, mi], 0]

    packed_ref[...] = rows_ref[...].reshape(tile_n, w)

    x = packed_ref[...]
    left = jax.lax.bitcast_convert_type(x & jnp.int32(-65536), jnp.float32)
    right = jax.lax.bitcast_convert_type(x << 16, jnp.float32)
    logits_ref[:, :w] = left
    logits_ref[:, w:] = right

    vals = logits_ref[...]
    col = jax.lax.broadcasted_iota(jnp.int32, (tile_n, v), 1)
    m = jnp.max(vals, axis=-1, keepdims=True)
    lse = m + jnp.log(jnp.sum(jnp.exp(vals - m), axis=-1, keepdims=True))
    tgt_logit = jnp.sum(jnp.where(col == tgt_ref[...], vals, 0.0),
                        axis=-1, keepdims=True)
    tokloss_ref[...] = lse - tgt_logit


def kernel(idx, embedding_table, targets):
    B, T = idx.shape
    V = embedding_table.shape[0]
    N = B * T
    W = V // 2

    tile_n = 1024
    assert N % tile_n == 0 and V % 256 == 0
    num_tiles = N // tile_n

    tb = embedding_table.astype(jnp.bfloat16)
    hi = jax.lax.bitcast_convert_type(tb[:, :W], jnp.uint16).astype(jnp.uint32)
    lo = jax.lax.bitcast_convert_type(tb[:, W:], jnp.uint16).astype(jnp.uint32)
    packed = jax.lax.bitcast_convert_type(
        (hi << 16) | lo, jnp.int32).reshape(V, 1, W)

    idx_rows = idx.reshape(num_tiles, 1, tile_n).astype(jnp.int32)
    tgt_col = targets.reshape(N, 1).astype(jnp.int32)

    body = functools.partial(_gather_ce_kernel, tile_n=tile_n, v=V)
    logits, tok_loss = pl.pallas_call(
        body,
        grid=(num_tiles,),
        out_shape=(
            jax.ShapeDtypeStruct((N, V), jnp.float32),
            jax.ShapeDtypeStruct((N, 1), jnp.float32),
        ),
        in_specs=[
            pl.BlockSpec((1, 1, tile_n), lambda i: (i, 0, 0),
                         memory_space=pltpu.SMEM),
            pl.BlockSpec((tile_n, 1), lambda i: (i, 0)),
            pl.BlockSpec((V, 1, W), lambda i: (0, 0, 0)),
        ],
        out_specs=(
            pl.BlockSpec((tile_n, V), lambda i: (i, 0)),
            pl.BlockSpec((tile_n, 1), lambda i: (i, 0)),
        ),
        scratch_shapes=[pltpu.VMEM((tile_n, 1, W), jnp.int32),
                        pltpu.VMEM((tile_n, W), jnp.int32)],
        compiler_params=pltpu.CompilerParams(
            dimension_semantics=("parallel",)),
    )(idx_rows, tgt_col, packed)

    loss = jnp.sum(tok_loss) / N
    return logits, loss
